# baseline (speedup 1.0000x reference)
import functools

import jax
import jax.numpy as jnp
from jax import lax
from jax.experimental import pallas as pl
from jax.experimental.pallas import tpu as pltpu
from jax.experimental.pallas import tpu_sc as plsc

N, K, D = 16384, 100000, 64
NC, NS, LANES = 2, 16, 16
NW = NC * NS
ROWS_PER_W = N // NW
CHUNK = 128
NCHUNK = ROWS_PER_W // CHUNK
HALF = D


def _sc_body(feat_hbm, idx_hbm, cb_hbm, ex_hbm, var_hbm, out_hbm,
             idx_v, cb_v, feat_v, ex_v, var_v, acc_v, sem):
    c = lax.axis_index("c")
    s = lax.axis_index("s")
    wid = s * NC + c

    pltpu.sync_copy(idx_hbm.at[wid], idx_v)
    pltpu.sync_copy(cb_hbm.at[wid], cb_v)

    zero = jnp.zeros((LANES,), jnp.float32)
    accs = (zero, zero, zero, zero)
    for j in range(NCHUNK):
        frow0 = pl.multiple_of((wid * ROWS_PER_W + j * CHUNK) // 2, 64)
        pltpu.sync_copy(feat_hbm.at[pl.ds(frow0, CHUNK // 2)], feat_v)
        pltpu.async_copy(ex_hbm.at[idx_v.at[j]], ex_v, sem).wait()
        pltpu.async_copy(var_hbm.at[idx_v.at[j]], var_v, sem).wait()

        def group_body(g, accs):
            a0, a1, a2, a3 = accs
            cbv = cb_v[j, pl.ds(g * LANES, LANES)]
            for u in range(LANES // 2):
                t = g * (LANES // 2) + u
                cb0 = cbv[2 * u]
                cb1 = cbv[2 * u + 1]
                for q in range(4):
                    f = feat_v[t, pl.ds(q * LANES, LANES)]
                    e = ex_v[2 * t, pl.ds(cb0 + q * LANES, LANES)]
                    v = var_v[2 * t, pl.ds(cb0 + q * LANES, LANES)]
                    if q == 0:
                        a0 = a0 + jnp.abs(f - e) / v
                    elif q == 1:
                        a1 = a1 + jnp.abs(f - e) / v
                    elif q == 2:
                        a2 = a2 + jnp.abs(f - e) / v
                    else:
                        a3 = a3 + jnp.abs(f - e) / v
                for q in range(4):
                    f = feat_v[t, pl.ds(HALF + q * LANES, LANES)]
                    e = ex_v[2 * t + 1, pl.ds(cb1 + q * LANES, LANES)]
                    v = var_v[2 * t + 1, pl.ds(cb1 + q * LANES, LANES)]
                    if q == 0:
                        a0 = a0 + jnp.abs(f - e) / v
                    elif q == 1:
                        a1 = a1 + jnp.abs(f - e) / v
                    elif q == 2:
                        a2 = a2 + jnp.abs(f - e) / v
                    else:
                        a3 = a3 + jnp.abs(f - e) / v
            return (a0, a1, a2, a3)

        accs = lax.fori_loop(0, CHUNK // LANES, group_body, accs)

    acc_v[...] = (accs[0] + accs[1]) + (accs[2] + accs[3])
    pltpu.sync_copy(acc_v, out_hbm.at[wid])


@jax.jit
def _sc_loss(feat2, idx3, cb3, ex2, var2):
    mesh = plsc.VectorSubcoreMesh(core_axis_name="c", subcore_axis_name="s")
    partials = pl.kernel(
        _sc_body,
        mesh=mesh,
        out_type=jax.ShapeDtypeStruct((NW, LANES), jnp.float32),
        compiler_params=pltpu.CompilerParams(use_tc_tiling_on_sc=True),
        scratch_types=[
            pltpu.VMEM((NCHUNK, CHUNK), jnp.int32),
            pltpu.VMEM((NCHUNK, CHUNK), jnp.int32),
            pltpu.VMEM((CHUNK // 2, 2 * D), jnp.float32),
            pltpu.VMEM((CHUNK, 2 * D), jnp.float32),
            pltpu.VMEM((CHUNK, 2 * D), jnp.float32),
            pltpu.VMEM((LANES,), jnp.float32),
            pltpu.SemaphoreType.DMA,
        ],
    )(feat2, idx3, cb3, ex2, var2)
    return jnp.sum(partials) / jnp.float32(N)


def kernel(action_features_actionframes, action_idxs_actionframes,
           examplars, examplars_variances):
    idx = action_idxs_actionframes.astype(jnp.int32)
    idx3 = (idx >> 1).reshape(NW, NCHUNK, CHUNK)
    cb3 = ((idx & 1) * HALF).reshape(NW, NCHUNK, CHUNK)
    feat2 = action_features_actionframes.reshape(N // 2, 2 * D)
    ex2 = examplars.reshape(K // 2, 2 * D)
    var2 = examplars_variances.reshape(K // 2, 2 * D)
    return _sc_loss(feat2, idx3, cb3, ex2, var2)


# speedup vs baseline: 1.1071x; 1.1071x over previous
import functools

import jax
import jax.numpy as jnp
from jax import lax
from jax.experimental import pallas as pl
from jax.experimental.pallas import tpu as pltpu
from jax.experimental.pallas import tpu_sc as plsc

N, K, D = 16384, 100000, 64
NC, NS, LANES = 2, 16, 16
NW = NC * NS
ROWS_PER_W = N // NW
CHUNK = 128
NCHUNK = ROWS_PER_W // CHUNK


def _sc_body(featT_hbm, idx_hbm, ex_hbm, var_hbm, out_hbm,
             idx_v, feat_v, ex_v, var_v, acc_v, sems):
    c = lax.axis_index("c")
    s = lax.axis_index("s")
    wid = s * NC + c
    base = wid * ROWS_PER_W

    pltpu.sync_copy(idx_hbm.at[pl.ds(base, ROWS_PER_W)], idx_v)
    pltpu.sync_copy(featT_hbm.at[:, pl.ds(base, ROWS_PER_W)], feat_v)

    def gathers(j, buf):
        isl = idx_v.at[pl.ds(j * CHUNK, CHUNK)]
        ce = pltpu.async_copy(ex_hbm.at[isl], ex_v.at[buf], sems.at[buf, 0])
        cv = pltpu.async_copy(var_hbm.at[isl], var_v.at[buf], sems.at[buf, 1])
        return ce, cv

    iota = lax.iota(jnp.int32, LANES)
    zero = jnp.zeros((LANES,), jnp.float32)
    accs = (zero, zero, zero, zero)
    pending = gathers(0, 0)
    for j in range(NCHUNK):
        pending[0].wait()
        pending[1].wait()
        if j + 1 < NCHUNK:
            pending = gathers(j + 1, (j + 1) % 2)
        buf = j % 2

        def row_body(r, accs):
            a0, a1, a2, a3 = accs
            p = jnp.full((LANES,), j * CHUNK + r, jnp.int32)
            f0 = plsc.load_gather(feat_v, [iota, p])
            e0 = ex_v[buf, r, pl.ds(0, LANES)]
            v0 = var_v[buf, r, pl.ds(0, LANES)]
            a0 = a0 + jnp.abs(f0 - e0) / v0
            f1 = plsc.load_gather(feat_v, [iota + LANES, p])
            e1 = ex_v[buf, r, pl.ds(LANES, LANES)]
            v1 = var_v[buf, r, pl.ds(LANES, LANES)]
            a1 = a1 + jnp.abs(f1 - e1) / v1
            f2 = plsc.load_gather(feat_v, [iota + 2 * LANES, p])
            e2 = ex_v[buf, r, pl.ds(2 * LANES, LANES)]
            v2 = var_v[buf, r, pl.ds(2 * LANES, LANES)]
            a2 = a2 + jnp.abs(f2 - e2) / v2
            f3 = plsc.load_gather(feat_v, [iota + 3 * LANES, p])
            e3 = ex_v[buf, r, pl.ds(3 * LANES, LANES)]
            v3 = var_v[buf, r, pl.ds(3 * LANES, LANES)]
            a3 = a3 + jnp.abs(f3 - e3) / v3
            return (a0, a1, a2, a3)

        accs = lax.fori_loop(0, CHUNK, row_body, accs)

    acc_v[...] = (accs[0] + accs[1]) + (accs[2] + accs[3])
    pltpu.sync_copy(acc_v, out_hbm.at[wid])


@jax.jit
def _sc_loss(featT, idx, ex, var):
    mesh = plsc.VectorSubcoreMesh(core_axis_name="c", subcore_axis_name="s")
    partials = pl.kernel(
        _sc_body,
        mesh=mesh,
        out_type=jax.ShapeDtypeStruct((NW, LANES), jnp.float32),
        compiler_params=pltpu.CompilerParams(
            use_tc_tiling_on_sc=False, needs_layout_passes=False),
        scratch_types=[
            pltpu.VMEM((ROWS_PER_W,), jnp.int32),
            pltpu.VMEM((D, ROWS_PER_W), jnp.float32),
            pltpu.VMEM((2, CHUNK, D), jnp.float32),
            pltpu.VMEM((2, CHUNK, D), jnp.float32),
            pltpu.VMEM((LANES,), jnp.float32),
            pltpu.SemaphoreType.DMA((2, 2)),
        ],
    )(featT, idx, ex, var)
    return jnp.sum(partials) / jnp.float32(N)


def kernel(action_features_actionframes, action_idxs_actionframes,
           examplars, examplars_variances):
    idx = action_idxs_actionframes.astype(jnp.int32)
    featT = action_features_actionframes.T
    return _sc_loss(featT, idx, examplars, examplars_variances)
